# Initial kernel scaffold; baseline (speedup 1.0000x reference)
import jax
import jax.numpy as jnp
from jax.experimental import pallas as pl


def kernel(logits, temperature, top_p, min_p, k):
    raise NotImplementedError("write your pallas kernel here")


# speedup vs baseline: 42.7940x; 42.7940x over previous
import jax
import jax.numpy as jnp
from jax.experimental import pallas as pl
from jax.experimental.pallas import tpu as pltpu

NEG_FILL = -1e30
TOPK = 50
ROWS = 8


def _sampler_block_kernel(logits_ref, temp_ref, top_p_ref, min_p_ref,
                          probs_ref, ids_ref):
    x = logits_ref[...] / temp_ref[...]
    v_dim = x.shape[-1]
    m = jnp.max(x, axis=-1, keepdims=True)

    lane = jax.lax.broadcasted_iota(jnp.int32, (1, TOPK), 1)

    def body(j, carry):
        cur, vals, cnts = carry
        mj = jnp.max(cur, axis=-1, keepdims=True)
        eq = cur == mj
        cj = jnp.sum(eq.astype(jnp.float32), axis=-1, keepdims=True)
        cur = jnp.where(eq, NEG_FILL, cur)
        sel = lane == j
        vals = jnp.where(sel, mj, vals)
        cnts = jnp.where(sel, cj, cnts)
        return cur, vals, cnts

    vals0 = jnp.zeros((x.shape[0], TOPK), dtype=jnp.float32)
    cnts0 = jnp.zeros((x.shape[0], TOPK), dtype=jnp.float32)
    _, vals, cnts = jax.lax.fori_loop(0, TOPK, body, (x, vals0, cnts0))

    e = jnp.exp(vals - m)
    ce = cnts * e
    ii = jax.lax.broadcasted_iota(jnp.int32, (TOPK, TOPK), 0)
    jj = jax.lax.broadcasted_iota(jnp.int32, (TOPK, TOPK), 1)
    tri = (ii <= jj).astype(jnp.float32)
    ccum = jax.lax.dot(cnts, tri, precision=jax.lax.Precision.HIGHEST)
    cprev = ccum - cnts
    topk_lvl = cprev < float(TOPK)
    ce_k = jnp.where(topk_lvl, ce, 0.0)
    s0 = jnp.sum(ce_k, axis=-1, keepdims=True)
    cum_e = jax.lax.dot(ce_k, tri, precision=jax.lax.Precision.HIGHEST)
    cum_e_prev = cum_e - ce_k
    keep_p = topk_lvl & (cum_e_prev < top_p_ref[...] * s0)
    thresh_p = jnp.min(jnp.where(keep_p, vals, jnp.inf), axis=-1, keepdims=True)
    thresh_mp = m + jnp.log(min_p_ref[...])
    thresh = jnp.maximum(thresh_p, thresh_mp)
    keep_final = topk_lvl & (vals >= thresh)
    s = jnp.sum(jnp.where(keep_final, ce, 0.0), axis=-1, keepdims=True)

    probs_ref[...] = jnp.where(x >= thresh, jnp.exp(x - m) / s, 0.0)
    idx = jax.lax.broadcasted_iota(jnp.int32, x.shape, 1)
    ids_ref[...] = jnp.min(jnp.where(x == m, idx, v_dim), axis=-1,
                           keepdims=True)


def kernel(logits, temperature, top_p, min_p, k):
    del k
    b, v = logits.shape
    grid = b // ROWS
    t2 = temperature.reshape(b, 1).astype(jnp.float32)
    p2 = top_p.reshape(b, 1).astype(jnp.float32)
    mp2 = min_p.reshape(b, 1).astype(jnp.float32)

    row_spec = pl.BlockSpec((ROWS, 1), lambda i: (i, 0))
    probs, ids = pl.pallas_call(
        _sampler_block_kernel,
        grid=(grid,),
        in_specs=[
            pl.BlockSpec((ROWS, v), lambda i: (i, 0)),
            row_spec, row_spec, row_spec,
        ],
        out_specs=[
            pl.BlockSpec((ROWS, v), lambda i: (i, 0)),
            pl.BlockSpec((ROWS, 1), lambda i: (i, 0)),
        ],
        out_shape=[
            jax.ShapeDtypeStruct((b, v), jnp.float32),
            jax.ShapeDtypeStruct((b, 1), jnp.int32),
        ],
        compiler_params=pltpu.CompilerParams(
            dimension_semantics=("arbitrary",),
        ),
    )(logits, t2, p2, mp2)
    return ids.reshape(b), probs
